# baseline (speedup 1.0000x reference)
import functools

import jax
import jax.numpy as jnp
from jax import lax
from jax.experimental import pallas as pl
from jax.experimental.pallas import tpu as pltpu
from jax.experimental.pallas import tpu_sc as plsc

_KS = 5
_K = _KS * _KS
_NC = 2
_NT = 16
_NW = _NC * _NT


def _prep_body(src_ref, dst_ref, eax_ref, eay_ref, meta_ref):
    px = eax_ref[...] * float(_KS - 1)
    py = eay_ref[...] * float(_KS - 1)
    ixc = jnp.clip(jnp.floor(px).astype(jnp.int32), 0, _KS - 2)
    iyc = jnp.clip(jnp.floor(py).astype(jnp.int32), 0, _KS - 2)
    gx = px - ixc.astype(jnp.float32)
    gy = py - iyc.astype(jnp.float32)
    meta_ref[0] = src_ref[...] * 16 + ixc + 4 * iyc
    meta_ref[1] = jax.lax.bitcast_convert_type((1.0 - gx) * (1.0 - gy),
                                               jnp.int32)
    meta_ref[2] = jax.lax.bitcast_convert_type(gx * (1.0 - gy), jnp.int32)
    meta_ref[3] = jax.lax.bitcast_convert_type((1.0 - gx) * gy, jnp.int32)
    meta_ref[4] = jax.lax.bitcast_convert_type(gx * gy, jnp.int32)
    meta_ref[5] = dst_ref[...]


def _edge_prep(src2, dst2, eax2, eay2):
    r, cb = src2.shape
    rb = 1000
    grid = r // rb
    return pl.pallas_call(
        _prep_body,
        grid=(grid,),
        in_specs=[
            pl.BlockSpec((rb, cb), lambda i: (i, 0)),
            pl.BlockSpec((rb, cb), lambda i: (i, 0)),
            pl.BlockSpec((rb, cb), lambda i: (i, 0)),
            pl.BlockSpec((rb, cb), lambda i: (i, 0)),
        ],
        out_specs=pl.BlockSpec((6, rb, cb), lambda i: (0, i, 0)),
        out_shape=jax.ShapeDtypeStruct((6, r, cb), jnp.int32),
    )(src2, dst2, eax2, eay2)


def _cell_weights(w):
    import numpy as np
    kidx = []
    for cell in range(16):
        cx, cy = cell % 4, cell // 4
        kidx += [cx + 5 * cy, cx + 1 + 5 * cy,
                 cx + 5 * (cy + 1), cx + 1 + 5 * (cy + 1)]
    wc = w[np.array(kidx)]
    din = w.shape[1]
    return jnp.transpose(wc, (1, 0, 2)).reshape(din, 64 * w.shape[2])


def _mm_body(a_ref, b_ref, o_ref):
    o_ref[...] = jax.lax.dot_general(
        a_ref[...], b_ref[...], (((1,), (0,)), ((), ())),
        preferred_element_type=jnp.float32)


def _matmul(a, b, block_rows):
    n, din = a.shape
    dout = b.shape[1]
    grid = n // block_rows
    return pl.pallas_call(
        _mm_body,
        grid=(grid,),
        in_specs=[
            pl.BlockSpec((block_rows, din), lambda i: (i, 0)),
            pl.BlockSpec((din, dout), lambda i: (0, 0)),
        ],
        out_specs=pl.BlockSpec((block_rows, dout), lambda i: (i, 0)),
        out_shape=jax.ShapeDtypeStruct((n, dout), jnp.float32),
    )(a, b)


def _make_sc_pass(n_nodes, o_dim, n_edges, with_cnt):
    ept = n_edges // _NW
    c = 80
    nch = ept // c
    gw = 4 * o_dim
    rpt = n_nodes // _NT
    zr = c

    mesh = plsc.VectorSubcoreMesh(core_axis_name="c", subcore_axis_name="s")
    out_type = [jax.ShapeDtypeStruct((_NC, n_nodes, o_dim), jnp.float32)]
    scratch = [
        pltpu.VMEM((8, 6, c), jnp.int32),
        pltpu.VMEM((3, c, gw), jnp.float32),
        pltpu.VMEM((2, c, o_dim), jnp.float32),
        pltpu.VMEM((2, c), jnp.int32),
        pltpu.VMEM_SHARED((n_nodes, o_dim), jnp.float32),
        pltpu.SemaphoreType.DMA((8,)),
        pltpu.SemaphoreType.DMA((3,)),
        pltpu.SemaphoreType.DMA((2,)),
    ]
    if with_cnt:
        out_type.append(jax.ShapeDtypeStruct((_NC, n_nodes, 16), jnp.float32))
        scratch += [
            pltpu.VMEM((c, 16), jnp.float32),
            pltpu.VMEM((zr, 16), jnp.float32),
            pltpu.VMEM_SHARED((n_nodes, 16), jnp.float32),
            pltpu.SemaphoreType.DMA((2,)),
        ]

    def body(xw, meta, *rest):
        if with_cnt:
            (agg_out, cnt_out, meta_m, rows_v, scv, dstc, agg_sh,
             sem_m, sem_g, sem_s,
             ones_v, z16_v, cnt_sh, sem_cs) = rest
        else:
            (agg_out, meta_m, rows_v, scv, dstc, agg_sh,
             sem_m, sem_g, sem_s) = rest
        cid = lax.axis_index("c")
        sid = lax.axis_index("s")
        wid = cid * _NT + sid
        row0 = sid * rpt

        @pl.loop(0, zr)
        def _fill_z(i):
            for j in range(o_dim // 16):
                scv[0, i, pl.ds(j * 16, 16)] = jnp.zeros((16,), jnp.float32)

        for r in range(rpt // zr):
            pltpu.sync_copy(scv.at[0], agg_sh.at[pl.ds(row0 + r * zr, zr)])

        if with_cnt:
            @pl.loop(0, zr)
            def _fill_z16(i):
                z16_v[i, :] = jnp.zeros((16,), jnp.float32)

            @pl.loop(0, c)
            def _fill_ones(i):
                ones_v[i, :] = jnp.ones((16,), jnp.float32)

            for r in range(rpt // zr):
                pltpu.sync_copy(z16_v, cnt_sh.at[pl.ds(row0 + r * zr, zr)])

        plsc.subcore_barrier()

        ebase = wid * ept

        def start_meta(g, b8):
            pltpu.async_copy(meta.at[:, pl.ds(ebase + g * c, c)],
                             meta_m.at[b8], sem_m.at[b8])

        def wait_meta(b8):
            pltpu.make_async_copy(meta.at[:, pl.ds(0, c)],
                                  meta_m.at[b8], sem_m.at[b8]).wait()

        def start_gather(b8, b3):
            pltpu.async_copy(xw.at[meta_m.at[b8, 0]], rows_v.at[b3],
                             sem_g.at[b3])

        def wait_gather(b3):
            pltpu.make_async_copy(xw.at[pl.ds(0, c)], rows_v.at[b3],
                                  sem_g.at[b3]).wait()

        def wait_scatter(b2):
            pltpu.make_async_copy(agg_out.at[0, pl.ds(0, c)], scv.at[b2],
                                  sem_s.at[b2]).wait()

        def wait_cnt_scatter(b2):
            pltpu.make_async_copy(cnt_out.at[0, pl.ds(0, c)], ones_v,
                                  sem_cs.at[b2]).wait()

        for k in range(8):
            start_meta(k, k)
        for k in range(2):
            wait_meta(k)
            start_gather(k, k)

        @pl.loop(0, nch)
        def _chunk(g):
            b2 = lax.rem(g, 2)
            b3 = lax.rem(g, 3)
            b8 = lax.rem(g, 8)
            wait_gather(b3)

            @pl.when(g >= 2)
            def _():
                wait_scatter(b2)
                if with_cnt:
                    wait_cnt_scatter(b2)

            for grp in range(c // 16):
                sl16 = pl.ds(grp * 16, 16)
                dstc[b2, sl16] = meta_m[b8, 5, sl16]
                s0 = plsc.bitcast(meta_m[b8, 1, sl16], jnp.float32)
                s1 = plsc.bitcast(meta_m[b8, 2, sl16], jnp.float32)
                s2 = plsc.bitcast(meta_m[b8, 3, sl16], jnp.float32)
                s3 = plsc.bitcast(meta_m[b8, 4, sl16], jnp.float32)
                for lane in range(16):
                    row = grp * 16 + lane
                    v0, v1, v2, v3 = s0[lane], s1[lane], s2[lane], s3[lane]
                    for j in range(o_dim // 16):
                        slj = pl.ds(j * 16, 16)
                        acc = rows_v[b3, row, pl.ds(j * 16, 16)] * v0
                        acc = acc + rows_v[b3, row, pl.ds(o_dim + j * 16, 16)] * v1
                        acc = acc + rows_v[b3, row, pl.ds(2 * o_dim + j * 16, 16)] * v2
                        acc = acc + rows_v[b3, row, pl.ds(3 * o_dim + j * 16, 16)] * v3
                        scv[b2, row, slj] = acc

            pltpu.async_copy(scv.at[b2], agg_sh.at[dstc.at[b2]],
                             sem_s.at[b2], add=True)
            if with_cnt:
                pltpu.async_copy(ones_v, cnt_sh.at[dstc.at[b2]],
                                 sem_cs.at[b2], add=True)

            @pl.when(g + 8 < nch)
            def _():
                start_meta(g + 8, b8)

            @pl.when(g + 2 < nch)
            def _():
                b8n = lax.rem(g + 2, 8)
                b3n = lax.rem(g + 2, 3)
                wait_meta(b8n)
                start_gather(b8n, b3n)

        for k in range(2):
            wait_scatter(k)
            if with_cnt:
                wait_cnt_scatter(k)

        plsc.subcore_barrier()

        for r in range(rpt // zr):
            sl = pl.ds(row0 + r * zr, zr)
            pltpu.sync_copy(agg_sh.at[sl], scv.at[0])
            pltpu.sync_copy(scv.at[0], agg_out.at[cid, sl])
        if with_cnt:
            for r in range(rpt // zr):
                sl = pl.ds(row0 + r * zr, zr)
                pltpu.sync_copy(cnt_sh.at[sl], z16_v)
                pltpu.sync_copy(z16_v, cnt_out.at[cid, sl])

    if not with_cnt:
        out_type = out_type[0]
    return pl.kernel(
        body, out_type, mesh=mesh, scratch_types=scratch,
        compiler_params=pltpu.CompilerParams(use_tc_tiling_on_sc=False,
                                             needs_layout_passes=False))


def _post1_body(agg_ref, cnt_ref, xp_ref, root_ref, b_ref, w2_ref,
                h_ref, xw2_ref):
    a = agg_ref[0] + agg_ref[1]
    cnt = cnt_ref[0, :, 0:1] + cnt_ref[1, :, 0:1]
    t = (a / jnp.maximum(cnt, 1.0)
         + jax.lax.dot_general(xp_ref[...], root_ref[...],
                               (((1,), (0,)), ((), ())),
                               preferred_element_type=jnp.float32)
         + b_ref[0:1, :])
    h = jnp.where(t > 0, t, jnp.exp(t) - 1.0)
    h_ref[...] = h
    xw2_ref[...] = jax.lax.dot_general(
        h, w2_ref[...], (((1,), (0,)), ((), ())),
        preferred_element_type=jnp.float32)


def _post1(agg1, cnt, xp, root1p, b1b, w2t, block_rows=1024):
    n = xp.shape[0]
    kd = w2t.shape[1]
    grid = n // block_rows
    return pl.pallas_call(
        _post1_body,
        grid=(grid,),
        in_specs=[
            pl.BlockSpec((2, block_rows, 32), lambda i: (0, i, 0)),
            pl.BlockSpec((2, block_rows, 16), lambda i: (0, i, 0)),
            pl.BlockSpec((block_rows, 8), lambda i: (i, 0)),
            pl.BlockSpec((8, 32), lambda i: (0, 0)),
            pl.BlockSpec((8, 32), lambda i: (0, 0)),
            pl.BlockSpec((32, kd), lambda i: (0, 0)),
        ],
        out_specs=[
            pl.BlockSpec((block_rows, 32), lambda i: (i, 0)),
            pl.BlockSpec((block_rows, kd), lambda i: (i, 0)),
        ],
        out_shape=[
            jax.ShapeDtypeStruct((n, 32), jnp.float32),
            jax.ShapeDtypeStruct((n, kd), jnp.float32),
        ],
    )(agg1, cnt, xp, root1p, b1b, w2t)


def _post2_body(agg_ref, cnt_ref, h1_ref, root_ref, b_ref, o_ref, *,
                block_rows, n_real):
    a = agg_ref[0] + agg_ref[1]
    cnt = cnt_ref[0, :, 0:1] + cnt_ref[1, :, 0:1]
    t = (a / jnp.maximum(cnt, 1.0)
         + jax.lax.dot_general(h1_ref[...], root_ref[...],
                               (((1,), (0,)), ((), ())),
                               preferred_element_type=jnp.float32)
         + b_ref[0:1, :])
    h2 = jnp.where(t > 0, t, jnp.exp(t) - 1.0)
    row = (pl.program_id(0) * block_rows
           + jax.lax.broadcasted_iota(jnp.int32, (block_rows, 1), 0))
    h2 = jnp.where(row < n_real, h2, 0.0)

    @pl.when(pl.program_id(0) == 0)
    def _():
        o_ref[...] = jnp.zeros_like(o_ref)

    o_ref[0:1, :] += jnp.sum(h2, axis=0, keepdims=True)


def _post2(agg2, cnt, h1, root2, b2b, n_real, block_rows=1024):
    n = h1.shape[0]
    grid = n // block_rows
    return pl.pallas_call(
        functools.partial(_post2_body, block_rows=block_rows, n_real=n_real),
        grid=(grid,),
        in_specs=[
            pl.BlockSpec((2, block_rows, 64), lambda i: (0, i, 0)),
            pl.BlockSpec((2, block_rows, 16), lambda i: (0, i, 0)),
            pl.BlockSpec((block_rows, 32), lambda i: (i, 0)),
            pl.BlockSpec((32, 64), lambda i: (0, 0)),
            pl.BlockSpec((8, 64), lambda i: (0, 0)),
        ],
        out_specs=pl.BlockSpec((8, 64), lambda i: (0, 0)),
        out_shape=jax.ShapeDtypeStruct((8, 64), jnp.float32),
    )(agg2, cnt, h1, root2, b2b)


def _final_body(g_ref, lw1_ref, lb1_ref, lw2_ref, lb2_ref, o_ref, *, n):
    g = jnp.sum(g_ref[...], axis=0, keepdims=True) * (1.0 / n)
    g8 = jnp.broadcast_to(g, (8, 64))
    t = jax.lax.dot_general(g8, lw1_ref[...], (((1,), (0,)), ((), ())),
                            preferred_element_type=jnp.float32)
    t = t + lb1_ref[0:1, :]
    t = jnp.where(t > 0, t, jnp.exp(t) - 1.0)
    lg = jax.lax.dot_general(t, lw2_ref[...], (((1,), (0,)), ((), ())),
                             preferred_element_type=jnp.float32)
    lg = lg + lb2_ref[0:1, :]
    l0 = lg[0:1, 0:1]
    shifted = l0 - l0
    res = shifted - jnp.log(jnp.sum(jnp.exp(shifted)))
    o_ref[...] = jnp.broadcast_to(res, (8, 128))


def _final(gsum8, lw1, lb1b, lw2p, lb2b, n):
    return pl.pallas_call(
        functools.partial(_final_body, n=n),
        out_shape=jax.ShapeDtypeStruct((8, 128), jnp.float32),
    )(gsum8, lw1, lb1b, lw2p, lb2b)


def kernel(x, edge_index, edge_attr, batch, W1, root1, b1, W2, root2, b2,
           lw1, lb1, lw2, lb2):
    n = x.shape[0]
    e = edge_index.shape[1]
    cb = 128
    r = e // cb

    src2 = edge_index[0].reshape(r, cb)
    dst = edge_index[1]
    dst2 = dst.reshape(r, cb)
    eax2 = edge_attr[:, 0].reshape(r, cb)
    eay2 = edge_attr[:, 1].reshape(r, cb)

    meta = _edge_prep(src2, dst2, eax2, eay2).reshape(6, e)

    npad = 10240
    xp = jnp.pad(x, ((0, npad - n), (0, 5)))
    w1c = jnp.pad(_cell_weights(W1), ((0, 5), (0, 0)))
    xw1 = _matmul(xp, w1c, 1024).reshape(npad * 16, 128)

    agg1, cnt = _make_sc_pass(npad, 32, e, True)(xw1, meta)

    root1p = jnp.pad(root1, ((0, 5), (0, 0)))
    b1b = jnp.broadcast_to(b1.reshape(1, 32), (8, 32))
    w2c = _cell_weights(W2)
    h1, xw2 = _post1(agg1, cnt, xp, root1p, b1b, w2c)
    xw2 = xw2.reshape(npad * 16, 256)

    agg2 = _make_sc_pass(npad, 64, e, False)(xw2, meta)

    b2b = jnp.broadcast_to(b2.reshape(1, 64), (8, 64))
    gsum8 = _post2(agg2, cnt, h1, root2, b2b, n)

    lb1b = jnp.broadcast_to(lb1.reshape(1, 128), (8, 128))
    lw2p = jnp.pad(lw2, ((0, 0), (0, 7)))
    lb2b = jnp.broadcast_to(lb2.reshape(1, 1), (8, 8))
    out = _final(gsum8, lw1, lb1b, lw2p, lb2b, n)
    return out[:1, :1]


# speedup vs baseline: 1.8287x; 1.8287x over previous
import functools

import jax
import jax.numpy as jnp
from jax import lax
from jax.experimental import pallas as pl
from jax.experimental.pallas import tpu as pltpu
from jax.experimental.pallas import tpu_sc as plsc

_KS = 5
_K = _KS * _KS
_NC = 2
_NT = 16
_NW = _NC * _NT


def _prep_body(src_ref, dst_ref, eax_ref, eay_ref, meta_ref):
    px = eax_ref[...] * float(_KS - 1)
    py = eay_ref[...] * float(_KS - 1)
    ixc = jnp.clip(jnp.floor(px).astype(jnp.int32), 0, _KS - 2)
    iyc = jnp.clip(jnp.floor(py).astype(jnp.int32), 0, _KS - 2)
    gx = px - ixc.astype(jnp.float32)
    gy = py - iyc.astype(jnp.float32)
    meta_ref[0] = src_ref[...] * 16 + ixc + 4 * iyc
    meta_ref[1] = jax.lax.bitcast_convert_type((1.0 - gx) * (1.0 - gy),
                                               jnp.int32)
    meta_ref[2] = jax.lax.bitcast_convert_type(gx * (1.0 - gy), jnp.int32)
    meta_ref[3] = jax.lax.bitcast_convert_type((1.0 - gx) * gy, jnp.int32)
    meta_ref[4] = jax.lax.bitcast_convert_type(gx * gy, jnp.int32)
    meta_ref[5] = dst_ref[...]


def _edge_prep(src2, dst2, eax2, eay2):
    r, cb = src2.shape
    rb = 1000
    grid = r // rb
    return pl.pallas_call(
        _prep_body,
        grid=(grid,),
        in_specs=[
            pl.BlockSpec((rb, cb), lambda i: (i, 0)),
            pl.BlockSpec((rb, cb), lambda i: (i, 0)),
            pl.BlockSpec((rb, cb), lambda i: (i, 0)),
            pl.BlockSpec((rb, cb), lambda i: (i, 0)),
        ],
        out_specs=pl.BlockSpec((6, rb, cb), lambda i: (0, i, 0)),
        out_shape=jax.ShapeDtypeStruct((6, r, cb), jnp.int32),
    )(src2, dst2, eax2, eay2)


def _cell_weights(w):
    import numpy as np
    kidx = []
    for cell in range(16):
        cx, cy = cell % 4, cell // 4
        kidx += [cx + 5 * cy, cx + 1 + 5 * cy,
                 cx + 5 * (cy + 1), cx + 1 + 5 * (cy + 1)]
    wc = w[np.array(kidx)]
    din = w.shape[1]
    return jnp.transpose(wc, (1, 0, 2)).reshape(din, 64 * w.shape[2])


def _mm_body(a_ref, b_ref, o_ref):
    o_ref[...] = jax.lax.dot_general(
        a_ref[...], b_ref[...], (((1,), (0,)), ((), ())),
        preferred_element_type=jnp.float32)


def _matmul(a, b, block_rows):
    n, din = a.shape
    dout = b.shape[1]
    grid = n // block_rows
    return pl.pallas_call(
        _mm_body,
        grid=(grid,),
        in_specs=[
            pl.BlockSpec((block_rows, din), lambda i: (i, 0)),
            pl.BlockSpec((din, dout), lambda i: (0, 0)),
        ],
        out_specs=pl.BlockSpec((block_rows, dout), lambda i: (i, 0)),
        out_shape=jax.ShapeDtypeStruct((n, dout), jnp.float32),
    )(a, b)


def _make_sc_pass(n_nodes, o_dim, n_edges, with_cnt):
    ept = n_edges // _NW
    c = 80
    nch = ept // c
    gw = 4 * o_dim
    rpt = n_nodes // _NT
    zr = c

    mesh = plsc.VectorSubcoreMesh(core_axis_name="c", subcore_axis_name="s")
    out_type = [jax.ShapeDtypeStruct((_NC, n_nodes, o_dim), jnp.float32)]
    scratch = [
        pltpu.VMEM((8, 6, c), jnp.int32),
        pltpu.VMEM((3, c, gw), jnp.float32),
        pltpu.VMEM((2, c, o_dim), jnp.float32),
        pltpu.VMEM((2, c), jnp.int32),
        pltpu.VMEM_SHARED((n_nodes, o_dim), jnp.float32),
        pltpu.SemaphoreType.DMA((8,)),
        pltpu.SemaphoreType.DMA((3,)),
        pltpu.SemaphoreType.DMA((2,)),
    ]
    if with_cnt:
        out_type.append(jax.ShapeDtypeStruct((_NC, n_nodes, 16), jnp.float32))
        scratch += [
            pltpu.VMEM((c, 16), jnp.float32),
            pltpu.VMEM((zr, 16), jnp.float32),
            pltpu.VMEM_SHARED((n_nodes, 16), jnp.float32),
            pltpu.SemaphoreType.DMA((2,)),
        ]

    def body(xw, meta, *rest):
        if with_cnt:
            (agg_out, cnt_out, meta_m, rows_v, scv, dstc, agg_sh,
             sem_m, sem_g, sem_s,
             ones_v, z16_v, cnt_sh, sem_cs) = rest
        else:
            (agg_out, meta_m, rows_v, scv, dstc, agg_sh,
             sem_m, sem_g, sem_s) = rest
        cid = lax.axis_index("c")
        sid = lax.axis_index("s")
        wid = cid * _NT + sid
        row0 = sid * rpt

        @pl.loop(0, zr)
        def _fill_z(i):
            for j in range(o_dim // 16):
                scv[0, i, pl.ds(j * 16, 16)] = jnp.zeros((16,), jnp.float32)

        for r in range(rpt // zr):
            pltpu.sync_copy(scv.at[0], agg_sh.at[pl.ds(row0 + r * zr, zr)])

        if with_cnt:
            @pl.loop(0, zr)
            def _fill_z16(i):
                z16_v[i, :] = jnp.zeros((16,), jnp.float32)

            @pl.loop(0, c)
            def _fill_ones(i):
                ones_v[i, :] = jnp.ones((16,), jnp.float32)

            for r in range(rpt // zr):
                pltpu.sync_copy(z16_v, cnt_sh.at[pl.ds(row0 + r * zr, zr)])

        plsc.subcore_barrier()

        ebase = wid * ept

        def start_meta(g, b8):
            pltpu.async_copy(meta.at[:, pl.ds(ebase + g * c, c)],
                             meta_m.at[b8], sem_m.at[b8])

        def wait_meta(b8):
            pltpu.make_async_copy(meta.at[:, pl.ds(0, c)],
                                  meta_m.at[b8], sem_m.at[b8]).wait()

        def start_gather(b8, b3):
            pltpu.async_copy(xw.at[meta_m.at[b8, 0]], rows_v.at[b3],
                             sem_g.at[b3])

        def wait_gather(b3):
            pltpu.make_async_copy(xw.at[pl.ds(0, c)], rows_v.at[b3],
                                  sem_g.at[b3]).wait()

        def wait_scatter(b2):
            pltpu.make_async_copy(agg_out.at[0, pl.ds(0, c)], scv.at[b2],
                                  sem_s.at[b2]).wait()

        def wait_cnt_scatter(b2):
            pltpu.make_async_copy(cnt_out.at[0, pl.ds(0, c)], ones_v,
                                  sem_cs.at[b2]).wait()

        for k in range(8):
            start_meta(k, k)
        for k in range(2):
            wait_meta(k)
            start_gather(k, k)

        @pl.loop(0, nch)
        def _chunk(g):
            b2 = lax.rem(g, 2)
            b3 = lax.rem(g, 3)
            b8 = lax.rem(g, 8)
            wait_gather(b3)

            @pl.when(g >= 2)
            def _():
                wait_scatter(b2)
                if with_cnt:
                    wait_cnt_scatter(b2)

            @plsc.parallel_loop(0, c // 16)
            def _combine(grp):
                sl16 = pl.ds(grp * 16, 16)
                dstc[b2, sl16] = meta_m[b8, 5, sl16]
                s0 = plsc.bitcast(meta_m[b8, 1, sl16], jnp.float32)
                s1 = plsc.bitcast(meta_m[b8, 2, sl16], jnp.float32)
                s2 = plsc.bitcast(meta_m[b8, 3, sl16], jnp.float32)
                s3 = plsc.bitcast(meta_m[b8, 4, sl16], jnp.float32)
                for lane in range(16):
                    row = grp * 16 + lane
                    v0, v1, v2, v3 = s0[lane], s1[lane], s2[lane], s3[lane]
                    for j in range(o_dim // 16):
                        slj = pl.ds(j * 16, 16)
                        acc = rows_v[b3, row, pl.ds(j * 16, 16)] * v0
                        acc = acc + rows_v[b3, row, pl.ds(o_dim + j * 16, 16)] * v1
                        acc = acc + rows_v[b3, row, pl.ds(2 * o_dim + j * 16, 16)] * v2
                        acc = acc + rows_v[b3, row, pl.ds(3 * o_dim + j * 16, 16)] * v3
                        scv[b2, row, slj] = acc

            pltpu.async_copy(scv.at[b2], agg_sh.at[dstc.at[b2]],
                             sem_s.at[b2], add=True)
            if with_cnt:
                pltpu.async_copy(ones_v, cnt_sh.at[dstc.at[b2]],
                                 sem_cs.at[b2], add=True)

            @pl.when(g + 8 < nch)
            def _():
                start_meta(g + 8, b8)

            @pl.when(g + 2 < nch)
            def _():
                b8n = lax.rem(g + 2, 8)
                b3n = lax.rem(g + 2, 3)
                wait_meta(b8n)
                start_gather(b8n, b3n)

        for k in range(2):
            wait_scatter(k)
            if with_cnt:
                wait_cnt_scatter(k)

        plsc.subcore_barrier()

        for r in range(rpt // zr):
            sl = pl.ds(row0 + r * zr, zr)
            pltpu.sync_copy(agg_sh.at[sl], scv.at[0])
            pltpu.sync_copy(scv.at[0], agg_out.at[cid, sl])
        if with_cnt:
            for r in range(rpt // zr):
                sl = pl.ds(row0 + r * zr, zr)
                pltpu.sync_copy(cnt_sh.at[sl], z16_v)
                pltpu.sync_copy(z16_v, cnt_out.at[cid, sl])

    if not with_cnt:
        out_type = out_type[0]
    return pl.kernel(
        body, out_type, mesh=mesh, scratch_types=scratch,
        compiler_params=pltpu.CompilerParams(use_tc_tiling_on_sc=False,
                                             needs_layout_passes=False))


def _post1_body(agg_ref, cnt_ref, xp_ref, root_ref, b_ref, w2_ref,
                h_ref, xw2_ref):
    a = agg_ref[0] + agg_ref[1]
    cnt = cnt_ref[0, :, 0:1] + cnt_ref[1, :, 0:1]
    t = (a / jnp.maximum(cnt, 1.0)
         + jax.lax.dot_general(xp_ref[...], root_ref[...],
                               (((1,), (0,)), ((), ())),
                               preferred_element_type=jnp.float32)
         + b_ref[0:1, :])
    h = jnp.where(t > 0, t, jnp.exp(t) - 1.0)
    h_ref[...] = h
    xw2_ref[...] = jax.lax.dot_general(
        h, w2_ref[...], (((1,), (0,)), ((), ())),
        preferred_element_type=jnp.float32)


def _post1(agg1, cnt, xp, root1p, b1b, w2t, block_rows=1024):
    n = xp.shape[0]
    kd = w2t.shape[1]
    grid = n // block_rows
    return pl.pallas_call(
        _post1_body,
        grid=(grid,),
        in_specs=[
            pl.BlockSpec((2, block_rows, 32), lambda i: (0, i, 0)),
            pl.BlockSpec((2, block_rows, 16), lambda i: (0, i, 0)),
            pl.BlockSpec((block_rows, 8), lambda i: (i, 0)),
            pl.BlockSpec((8, 32), lambda i: (0, 0)),
            pl.BlockSpec((8, 32), lambda i: (0, 0)),
            pl.BlockSpec((32, kd), lambda i: (0, 0)),
        ],
        out_specs=[
            pl.BlockSpec((block_rows, 32), lambda i: (i, 0)),
            pl.BlockSpec((block_rows, kd), lambda i: (i, 0)),
        ],
        out_shape=[
            jax.ShapeDtypeStruct((n, 32), jnp.float32),
            jax.ShapeDtypeStruct((n, kd), jnp.float32),
        ],
    )(agg1, cnt, xp, root1p, b1b, w2t)


def _post2_body(agg_ref, cnt_ref, h1_ref, root_ref, b_ref, o_ref, *,
                block_rows, n_real):
    a = agg_ref[0] + agg_ref[1]
    cnt = cnt_ref[0, :, 0:1] + cnt_ref[1, :, 0:1]
    t = (a / jnp.maximum(cnt, 1.0)
         + jax.lax.dot_general(h1_ref[...], root_ref[...],
                               (((1,), (0,)), ((), ())),
                               preferred_element_type=jnp.float32)
         + b_ref[0:1, :])
    h2 = jnp.where(t > 0, t, jnp.exp(t) - 1.0)
    row = (pl.program_id(0) * block_rows
           + jax.lax.broadcasted_iota(jnp.int32, (block_rows, 1), 0))
    h2 = jnp.where(row < n_real, h2, 0.0)

    @pl.when(pl.program_id(0) == 0)
    def _():
        o_ref[...] = jnp.zeros_like(o_ref)

    o_ref[0:1, :] += jnp.sum(h2, axis=0, keepdims=True)


def _post2(agg2, cnt, h1, root2, b2b, n_real, block_rows=1024):
    n = h1.shape[0]
    grid = n // block_rows
    return pl.pallas_call(
        functools.partial(_post2_body, block_rows=block_rows, n_real=n_real),
        grid=(grid,),
        in_specs=[
            pl.BlockSpec((2, block_rows, 64), lambda i: (0, i, 0)),
            pl.BlockSpec((2, block_rows, 16), lambda i: (0, i, 0)),
            pl.BlockSpec((block_rows, 32), lambda i: (i, 0)),
            pl.BlockSpec((32, 64), lambda i: (0, 0)),
            pl.BlockSpec((8, 64), lambda i: (0, 0)),
        ],
        out_specs=pl.BlockSpec((8, 64), lambda i: (0, 0)),
        out_shape=jax.ShapeDtypeStruct((8, 64), jnp.float32),
    )(agg2, cnt, h1, root2, b2b)


def _final_body(g_ref, lw1_ref, lb1_ref, lw2_ref, lb2_ref, o_ref, *, n):
    g = jnp.sum(g_ref[...], axis=0, keepdims=True) * (1.0 / n)
    g8 = jnp.broadcast_to(g, (8, 64))
    t = jax.lax.dot_general(g8, lw1_ref[...], (((1,), (0,)), ((), ())),
                            preferred_element_type=jnp.float32)
    t = t + lb1_ref[0:1, :]
    t = jnp.where(t > 0, t, jnp.exp(t) - 1.0)
    lg = jax.lax.dot_general(t, lw2_ref[...], (((1,), (0,)), ((), ())),
                             preferred_element_type=jnp.float32)
    lg = lg + lb2_ref[0:1, :]
    l0 = lg[0:1, 0:1]
    shifted = l0 - l0
    res = shifted - jnp.log(jnp.sum(jnp.exp(shifted)))
    o_ref[...] = jnp.broadcast_to(res, (8, 128))


def _final(gsum8, lw1, lb1b, lw2p, lb2b, n):
    return pl.pallas_call(
        functools.partial(_final_body, n=n),
        out_shape=jax.ShapeDtypeStruct((8, 128), jnp.float32),
    )(gsum8, lw1, lb1b, lw2p, lb2b)


def kernel(x, edge_index, edge_attr, batch, W1, root1, b1, W2, root2, b2,
           lw1, lb1, lw2, lb2):
    n = x.shape[0]
    e = edge_index.shape[1]
    cb = 128
    r = e // cb

    src2 = edge_index[0].reshape(r, cb)
    dst = edge_index[1]
    dst2 = dst.reshape(r, cb)
    eax2 = edge_attr[:, 0].reshape(r, cb)
    eay2 = edge_attr[:, 1].reshape(r, cb)

    meta = _edge_prep(src2, dst2, eax2, eay2).reshape(6, e)

    npad = 10240
    xp = jnp.pad(x, ((0, npad - n), (0, 5)))
    w1c = jnp.pad(_cell_weights(W1), ((0, 5), (0, 0)))
    xw1 = _matmul(xp, w1c, 1024).reshape(npad * 16, 128)

    agg1, cnt = _make_sc_pass(npad, 32, e, True)(xw1, meta)

    root1p = jnp.pad(root1, ((0, 5), (0, 0)))
    b1b = jnp.broadcast_to(b1.reshape(1, 32), (8, 32))
    w2c = _cell_weights(W2)
    h1, xw2 = _post1(agg1, cnt, xp, root1p, b1b, w2c)
    xw2 = xw2.reshape(npad * 16, 256)

    agg2 = _make_sc_pass(npad, 64, e, False)(xw2, meta)

    b2b = jnp.broadcast_to(b2.reshape(1, 64), (8, 64))
    gsum8 = _post2(agg2, cnt, h1, root2, b2b, n)

    lb1b = jnp.broadcast_to(lb1.reshape(1, 128), (8, 128))
    lw2p = jnp.pad(lw2, ((0, 0), (0, 7)))
    lb2b = jnp.broadcast_to(lb2.reshape(1, 1), (8, 8))
    out = _final(gsum8, lw1, lb1b, lw2p, lb2b, n)
    return out[:1, :1]


# speedup vs baseline: 2.0257x; 1.1077x over previous
import functools

import jax
import jax.numpy as jnp
from jax import lax
from jax.experimental import pallas as pl
from jax.experimental.pallas import tpu as pltpu
from jax.experimental.pallas import tpu_sc as plsc

_KS = 5
_K = _KS * _KS
_NC = 2
_NT = 16
_NW = _NC * _NT


def _prep_body(src_ref, dst_ref, eax_ref, eay_ref, meta_ref):
    px = eax_ref[...] * float(_KS - 1)
    py = eay_ref[...] * float(_KS - 1)
    ixc = jnp.clip(jnp.floor(px).astype(jnp.int32), 0, _KS - 2)
    iyc = jnp.clip(jnp.floor(py).astype(jnp.int32), 0, _KS - 2)
    gx = px - ixc.astype(jnp.float32)
    gy = py - iyc.astype(jnp.float32)
    meta_ref[0] = src_ref[...] * 16 + ixc + 4 * iyc
    meta_ref[1] = jax.lax.bitcast_convert_type((1.0 - gx) * (1.0 - gy),
                                               jnp.int32)
    meta_ref[2] = jax.lax.bitcast_convert_type(gx * (1.0 - gy), jnp.int32)
    meta_ref[3] = jax.lax.bitcast_convert_type((1.0 - gx) * gy, jnp.int32)
    meta_ref[4] = jax.lax.bitcast_convert_type(gx * gy, jnp.int32)
    meta_ref[5] = dst_ref[...]


def _edge_prep(src2, dst2, eax2, eay2):
    r, cb = src2.shape
    rb = 1000
    grid = r // rb
    return pl.pallas_call(
        _prep_body,
        grid=(grid,),
        in_specs=[
            pl.BlockSpec((rb, cb), lambda i: (i, 0)),
            pl.BlockSpec((rb, cb), lambda i: (i, 0)),
            pl.BlockSpec((rb, cb), lambda i: (i, 0)),
            pl.BlockSpec((rb, cb), lambda i: (i, 0)),
        ],
        out_specs=pl.BlockSpec((6, rb, cb), lambda i: (0, i, 0)),
        out_shape=jax.ShapeDtypeStruct((6, r, cb), jnp.int32),
    )(src2, dst2, eax2, eay2)


def _cell_weights(w):
    import numpy as np
    kidx = []
    for cell in range(16):
        cx, cy = cell % 4, cell // 4
        kidx += [cx + 5 * cy, cx + 1 + 5 * cy,
                 cx + 5 * (cy + 1), cx + 1 + 5 * (cy + 1)]
    wc = w[np.array(kidx)]
    din = w.shape[1]
    flat = jnp.transpose(wc, (1, 0, 2)).reshape(din, 64 * w.shape[2])
    width = 64 * w.shape[2]
    src = np.empty((width,), np.int64)
    for g0 in range(0, width, 32):
        for i in range(16):
            src[g0 + 2 * i] = g0 + i
            src[g0 + 2 * i + 1] = g0 + 16 + i
    return flat[:, src]


def _mm_body(a_ref, b_ref, o_ref):
    o_ref[...] = jax.lax.dot_general(
        a_ref[...], b_ref[...], (((1,), (0,)), ((), ())),
        preferred_element_type=jnp.float32).astype(jnp.bfloat16)


def _matmul(a, b, block_rows):
    n, din = a.shape
    dout = b.shape[1]
    grid = n // block_rows
    return pl.pallas_call(
        _mm_body,
        grid=(grid,),
        in_specs=[
            pl.BlockSpec((block_rows, din), lambda i: (i, 0)),
            pl.BlockSpec((din, dout), lambda i: (0, 0)),
        ],
        out_specs=pl.BlockSpec((block_rows, dout), lambda i: (i, 0)),
        out_shape=jax.ShapeDtypeStruct((n, dout), jnp.bfloat16),
    )(a, b)


def _make_sc_pass(n_nodes, o_dim, n_edges, with_cnt):
    ept = n_edges // _NW
    c = 80
    nch = ept // c
    gw = 4 * o_dim
    rpt = n_nodes // _NT
    zr = c

    mesh = plsc.VectorSubcoreMesh(core_axis_name="c", subcore_axis_name="s")
    out_type = [jax.ShapeDtypeStruct((_NC, n_nodes, o_dim), jnp.float32)]
    scratch = [
        pltpu.VMEM((8, 6, c), jnp.int32),
        pltpu.VMEM((3, c, gw), jnp.bfloat16),
        pltpu.VMEM((2, c, o_dim), jnp.float32),
        pltpu.VMEM((2, c), jnp.int32),
        pltpu.VMEM_SHARED((n_nodes, o_dim), jnp.float32),
        pltpu.SemaphoreType.DMA((8,)),
        pltpu.SemaphoreType.DMA((3,)),
        pltpu.SemaphoreType.DMA((2,)),
    ]
    if with_cnt:
        out_type.append(jax.ShapeDtypeStruct((_NC, n_nodes, 16), jnp.float32))
        scratch += [
            pltpu.VMEM((c, 16), jnp.float32),
            pltpu.VMEM((zr, 16), jnp.float32),
            pltpu.VMEM_SHARED((n_nodes, 16), jnp.float32),
            pltpu.SemaphoreType.DMA((2,)),
        ]

    def body(xw, meta, *rest):
        if with_cnt:
            (agg_out, cnt_out, meta_m, rows_v, scv, dstc, agg_sh,
             sem_m, sem_g, sem_s,
             ones_v, z16_v, cnt_sh, sem_cs) = rest
        else:
            (agg_out, meta_m, rows_v, scv, dstc, agg_sh,
             sem_m, sem_g, sem_s) = rest
        cid = lax.axis_index("c")
        sid = lax.axis_index("s")
        wid = cid * _NT + sid
        row0 = sid * rpt

        @pl.loop(0, zr)
        def _fill_z(i):
            for j in range(o_dim // 16):
                scv[0, i, pl.ds(j * 16, 16)] = jnp.zeros((16,), jnp.float32)

        for r in range(rpt // zr):
            pltpu.sync_copy(scv.at[0], agg_sh.at[pl.ds(row0 + r * zr, zr)])

        if with_cnt:
            @pl.loop(0, zr)
            def _fill_z16(i):
                z16_v[i, :] = jnp.zeros((16,), jnp.float32)

            @pl.loop(0, c)
            def _fill_ones(i):
                ones_v[i, :] = jnp.ones((16,), jnp.float32)

            for r in range(rpt // zr):
                pltpu.sync_copy(z16_v, cnt_sh.at[pl.ds(row0 + r * zr, zr)])

        plsc.subcore_barrier()

        ebase = wid * ept

        def start_meta(g, b8):
            pltpu.async_copy(meta.at[:, pl.ds(ebase + g * c, c)],
                             meta_m.at[b8], sem_m.at[b8])

        def wait_meta(b8):
            pltpu.make_async_copy(meta.at[:, pl.ds(0, c)],
                                  meta_m.at[b8], sem_m.at[b8]).wait()

        def start_gather(b8, b3):
            pltpu.async_copy(xw.at[meta_m.at[b8, 0]], rows_v.at[b3],
                             sem_g.at[b3])

        def wait_gather(b3):
            pltpu.make_async_copy(xw.at[pl.ds(0, c)], rows_v.at[b3],
                                  sem_g.at[b3]).wait()

        def wait_scatter(b2):
            pltpu.make_async_copy(agg_out.at[0, pl.ds(0, c)], scv.at[b2],
                                  sem_s.at[b2]).wait()

        def wait_cnt_scatter(b2):
            pltpu.make_async_copy(cnt_out.at[0, pl.ds(0, c)], ones_v,
                                  sem_cs.at[b2]).wait()

        for k in range(8):
            start_meta(k, k)
        for k in range(2):
            wait_meta(k)
            start_gather(k, k)

        @pl.loop(0, nch)
        def _chunk(g):
            b2 = lax.rem(g, 2)
            b3 = lax.rem(g, 3)
            b8 = lax.rem(g, 8)
            wait_gather(b3)

            @pl.when(g >= 2)
            def _():
                wait_scatter(b2)
                if with_cnt:
                    wait_cnt_scatter(b2)

            @plsc.parallel_loop(0, c // 16)
            def _combine(grp):
                sl16 = pl.ds(grp * 16, 16)
                dstc[b2, sl16] = meta_m[b8, 5, sl16]
                s0 = plsc.bitcast(meta_m[b8, 1, sl16], jnp.float32)
                s1 = plsc.bitcast(meta_m[b8, 2, sl16], jnp.float32)
                s2 = plsc.bitcast(meta_m[b8, 3, sl16], jnp.float32)
                s3 = plsc.bitcast(meta_m[b8, 4, sl16], jnp.float32)
                vs = (s0, s1, s2, s3)
                for lane in range(16):
                    row = grp * 16 + lane
                    for m in range(o_dim // 32):
                        acc_a = None
                        acc_b = None
                        for s in range(4):
                            pk = rows_v[b3, row,
                                        pl.ds(s * o_dim + m * 32, 32)]
                            ua, ub = plsc.unpack(
                                pk, format=plsc.PackFormat.INTERLEAVED,
                                preferred_element_type=jnp.float32)
                            sval = vs[s][lane]
                            if acc_a is None:
                                acc_a = ua * sval
                                acc_b = ub * sval
                            else:
                                acc_a = acc_a + ua * sval
                                acc_b = acc_b + ub * sval
                        scv[b2, row, pl.ds(m * 32, 16)] = acc_a
                        scv[b2, row, pl.ds(m * 32 + 16, 16)] = acc_b

            pltpu.async_copy(scv.at[b2], agg_sh.at[dstc.at[b2]],
                             sem_s.at[b2], add=True)
            if with_cnt:
                pltpu.async_copy(ones_v, cnt_sh.at[dstc.at[b2]],
                                 sem_cs.at[b2], add=True)

            @pl.when(g + 8 < nch)
            def _():
                start_meta(g + 8, b8)

            @pl.when(g + 2 < nch)
            def _():
                b8n = lax.rem(g + 2, 8)
                b3n = lax.rem(g + 2, 3)
                wait_meta(b8n)
                start_gather(b8n, b3n)

        for k in range(2):
            wait_scatter(k)
            if with_cnt:
                wait_cnt_scatter(k)

        plsc.subcore_barrier()

        for r in range(rpt // zr):
            sl = pl.ds(row0 + r * zr, zr)
            pltpu.sync_copy(agg_sh.at[sl], scv.at[0])
            pltpu.sync_copy(scv.at[0], agg_out.at[cid, sl])
        if with_cnt:
            for r in range(rpt // zr):
                sl = pl.ds(row0 + r * zr, zr)
                pltpu.sync_copy(cnt_sh.at[sl], z16_v)
                pltpu.sync_copy(z16_v, cnt_out.at[cid, sl])

    if not with_cnt:
        out_type = out_type[0]
    return pl.kernel(
        body, out_type, mesh=mesh, scratch_types=scratch,
        compiler_params=pltpu.CompilerParams(use_tc_tiling_on_sc=False,
                                             needs_layout_passes=False))


def _post1_body(agg_ref, cnt_ref, xp_ref, root_ref, b_ref, w2_ref,
                h_ref, xw2_ref):
    a = agg_ref[0] + agg_ref[1]
    cnt = cnt_ref[0, :, 0:1] + cnt_ref[1, :, 0:1]
    t = (a / jnp.maximum(cnt, 1.0)
         + jax.lax.dot_general(xp_ref[...], root_ref[...],
                               (((1,), (0,)), ((), ())),
                               preferred_element_type=jnp.float32)
         + b_ref[0:1, :])
    h = jnp.where(t > 0, t, jnp.exp(t) - 1.0)
    h_ref[...] = h
    xw2_ref[...] = jax.lax.dot_general(
        h, w2_ref[...], (((1,), (0,)), ((), ())),
        preferred_element_type=jnp.float32).astype(jnp.bfloat16)


def _post1(agg1, cnt, xp, root1p, b1b, w2t, block_rows=1024):
    n = xp.shape[0]
    kd = w2t.shape[1]
    grid = n // block_rows
    return pl.pallas_call(
        _post1_body,
        grid=(grid,),
        in_specs=[
            pl.BlockSpec((2, block_rows, 32), lambda i: (0, i, 0)),
            pl.BlockSpec((2, block_rows, 16), lambda i: (0, i, 0)),
            pl.BlockSpec((block_rows, 8), lambda i: (i, 0)),
            pl.BlockSpec((8, 32), lambda i: (0, 0)),
            pl.BlockSpec((8, 32), lambda i: (0, 0)),
            pl.BlockSpec((32, kd), lambda i: (0, 0)),
        ],
        out_specs=[
            pl.BlockSpec((block_rows, 32), lambda i: (i, 0)),
            pl.BlockSpec((block_rows, kd), lambda i: (i, 0)),
        ],
        out_shape=[
            jax.ShapeDtypeStruct((n, 32), jnp.float32),
            jax.ShapeDtypeStruct((n, kd), jnp.bfloat16),
        ],
    )(agg1, cnt, xp, root1p, b1b, w2t)


def _post2_body(agg_ref, cnt_ref, h1_ref, root_ref, b_ref, o_ref, *,
                block_rows, n_real):
    a = agg_ref[0] + agg_ref[1]
    cnt = cnt_ref[0, :, 0:1] + cnt_ref[1, :, 0:1]
    t = (a / jnp.maximum(cnt, 1.0)
         + jax.lax.dot_general(h1_ref[...], root_ref[...],
                               (((1,), (0,)), ((), ())),
                               preferred_element_type=jnp.float32)
         + b_ref[0:1, :])
    h2 = jnp.where(t > 0, t, jnp.exp(t) - 1.0)
    row = (pl.program_id(0) * block_rows
           + jax.lax.broadcasted_iota(jnp.int32, (block_rows, 1), 0))
    h2 = jnp.where(row < n_real, h2, 0.0)

    @pl.when(pl.program_id(0) == 0)
    def _():
        o_ref[...] = jnp.zeros_like(o_ref)

    o_ref[0:1, :] += jnp.sum(h2, axis=0, keepdims=True)


def _post2(agg2, cnt, h1, root2, b2b, n_real, block_rows=1024):
    n = h1.shape[0]
    grid = n // block_rows
    return pl.pallas_call(
        functools.partial(_post2_body, block_rows=block_rows, n_real=n_real),
        grid=(grid,),
        in_specs=[
            pl.BlockSpec((2, block_rows, 64), lambda i: (0, i, 0)),
            pl.BlockSpec((2, block_rows, 16), lambda i: (0, i, 0)),
            pl.BlockSpec((block_rows, 32), lambda i: (i, 0)),
            pl.BlockSpec((32, 64), lambda i: (0, 0)),
            pl.BlockSpec((8, 64), lambda i: (0, 0)),
        ],
        out_specs=pl.BlockSpec((8, 64), lambda i: (0, 0)),
        out_shape=jax.ShapeDtypeStruct((8, 64), jnp.float32),
    )(agg2, cnt, h1, root2, b2b)


def _final_body(g_ref, lw1_ref, lb1_ref, lw2_ref, lb2_ref, o_ref, *, n):
    g = jnp.sum(g_ref[...], axis=0, keepdims=True) * (1.0 / n)
    g8 = jnp.broadcast_to(g, (8, 64))
    t = jax.lax.dot_general(g8, lw1_ref[...], (((1,), (0,)), ((), ())),
                            preferred_element_type=jnp.float32)
    t = t + lb1_ref[0:1, :]
    t = jnp.where(t > 0, t, jnp.exp(t) - 1.0)
    lg = jax.lax.dot_general(t, lw2_ref[...], (((1,), (0,)), ((), ())),
                             preferred_element_type=jnp.float32)
    lg = lg + lb2_ref[0:1, :]
    l0 = lg[0:1, 0:1]
    shifted = l0 - l0
    res = shifted - jnp.log(jnp.sum(jnp.exp(shifted)))
    o_ref[...] = jnp.broadcast_to(res, (8, 128))


def _final(gsum8, lw1, lb1b, lw2p, lb2b, n):
    return pl.pallas_call(
        functools.partial(_final_body, n=n),
        out_shape=jax.ShapeDtypeStruct((8, 128), jnp.float32),
    )(gsum8, lw1, lb1b, lw2p, lb2b)


def kernel(x, edge_index, edge_attr, batch, W1, root1, b1, W2, root2, b2,
           lw1, lb1, lw2, lb2):
    n = x.shape[0]
    e = edge_index.shape[1]
    cb = 128
    r = e // cb

    src2 = edge_index[0].reshape(r, cb)
    dst = edge_index[1]
    dst2 = dst.reshape(r, cb)
    eax2 = edge_attr[:, 0].reshape(r, cb)
    eay2 = edge_attr[:, 1].reshape(r, cb)

    meta = _edge_prep(src2, dst2, eax2, eay2).reshape(6, e)

    npad = 10240
    xp = jnp.pad(x, ((0, npad - n), (0, 5)))
    w1c = jnp.pad(_cell_weights(W1), ((0, 5), (0, 0)))
    xw1 = _matmul(xp, w1c, 1024).reshape(npad * 16, 128)

    agg1, cnt = _make_sc_pass(npad, 32, e, True)(xw1, meta)

    root1p = jnp.pad(root1, ((0, 5), (0, 0)))
    b1b = jnp.broadcast_to(b1.reshape(1, 32), (8, 32))
    w2c = _cell_weights(W2)
    h1, xw2 = _post1(agg1, cnt, xp, root1p, b1b, w2c)
    xw2 = xw2.reshape(npad * 16, 256)

    agg2 = _make_sc_pass(npad, 64, e, False)(xw2, meta)

    b2b = jnp.broadcast_to(b2.reshape(1, 64), (8, 64))
    gsum8 = _post2(agg2, cnt, h1, root2, b2b, n)

    lb1b = jnp.broadcast_to(lb1.reshape(1, 128), (8, 128))
    lw2p = jnp.pad(lw2, ((0, 0), (0, 7)))
    lb2b = jnp.broadcast_to(lb2.reshape(1, 1), (8, 8))
    out = _final(gsum8, lw1, lb1b, lw2p, lb2b, n)
    return out[:1, :1]


# speedup vs baseline: 2.1699x; 1.0712x over previous
import functools

import jax
import jax.numpy as jnp
from jax import lax
from jax.experimental import pallas as pl
from jax.experimental.pallas import tpu as pltpu
from jax.experimental.pallas import tpu_sc as plsc

_KS = 5
_K = _KS * _KS
_NC = 2
_NT = 16
_NW = _NC * _NT


def _prep_body(src_ref, dst_ref, eax_ref, eay_ref, meta_ref):
    px = eax_ref[...] * float(_KS - 1)
    py = eay_ref[...] * float(_KS - 1)
    ixc = jnp.clip(jnp.floor(px).astype(jnp.int32), 0, _KS - 2)
    iyc = jnp.clip(jnp.floor(py).astype(jnp.int32), 0, _KS - 2)
    gx = px - ixc.astype(jnp.float32)
    gy = py - iyc.astype(jnp.float32)
    meta_ref[0] = src_ref[...] * 16 + ixc + 4 * iyc
    meta_ref[1] = jax.lax.bitcast_convert_type((1.0 - gx) * (1.0 - gy),
                                               jnp.int32)
    meta_ref[2] = jax.lax.bitcast_convert_type(gx * (1.0 - gy), jnp.int32)
    meta_ref[3] = jax.lax.bitcast_convert_type((1.0 - gx) * gy, jnp.int32)
    meta_ref[4] = jax.lax.bitcast_convert_type(gx * gy, jnp.int32)
    meta_ref[5] = dst_ref[...]


def _edge_prep(src2, dst2, eax2, eay2):
    r, cb = src2.shape
    rb = 1000
    grid = r // rb
    return pl.pallas_call(
        _prep_body,
        grid=(grid,),
        in_specs=[
            pl.BlockSpec((rb, cb), lambda i: (i, 0)),
            pl.BlockSpec((rb, cb), lambda i: (i, 0)),
            pl.BlockSpec((rb, cb), lambda i: (i, 0)),
            pl.BlockSpec((rb, cb), lambda i: (i, 0)),
        ],
        out_specs=pl.BlockSpec((6, rb, cb), lambda i: (0, i, 0)),
        out_shape=jax.ShapeDtypeStruct((6, r, cb), jnp.int32),
    )(src2, dst2, eax2, eay2)


def _cell_weights(w):
    import numpy as np
    kidx = []
    for cell in range(16):
        cx, cy = cell % 4, cell // 4
        kidx += [cx + 5 * cy, cx + 1 + 5 * cy,
                 cx + 5 * (cy + 1), cx + 1 + 5 * (cy + 1)]
    wc = w[np.array(kidx)]
    din = w.shape[1]
    flat = jnp.transpose(wc, (1, 0, 2)).reshape(din, 64 * w.shape[2])
    width = 64 * w.shape[2]
    src = np.empty((width,), np.int64)
    for g0 in range(0, width, 32):
        for i in range(16):
            src[g0 + 2 * i] = g0 + i
            src[g0 + 2 * i + 1] = g0 + 16 + i
    return flat[:, src]


def _mm_body(a_ref, b_ref, o_ref):
    o_ref[...] = jax.lax.dot_general(
        a_ref[...], b_ref[...], (((1,), (0,)), ((), ())),
        preferred_element_type=jnp.float32).astype(jnp.bfloat16)


def _matmul(a, b, block_rows):
    n, din = a.shape
    dout = b.shape[1]
    grid = n // block_rows
    return pl.pallas_call(
        _mm_body,
        grid=(grid,),
        in_specs=[
            pl.BlockSpec((block_rows, din), lambda i: (i, 0)),
            pl.BlockSpec((din, dout), lambda i: (0, 0)),
        ],
        out_specs=pl.BlockSpec((block_rows, dout), lambda i: (i, 0)),
        out_shape=jax.ShapeDtypeStruct((n, dout), jnp.bfloat16),
    )(a, b)


def _make_sc_pass(n_nodes, o_dim, n_edges, with_cnt):
    ept = n_edges // _NW
    c = 80
    nch = ept // c
    gw = 4 * o_dim
    rpt = n_nodes // _NT
    zr = c

    mesh = plsc.VectorSubcoreMesh(core_axis_name="c", subcore_axis_name="s")
    out_type = [jax.ShapeDtypeStruct((_NC, n_nodes, o_dim), jnp.float32)]
    scratch = [
        pltpu.VMEM((8, 6, c), jnp.int32),
        pltpu.VMEM((4, c, gw), jnp.bfloat16),
        pltpu.VMEM((3, c, o_dim), jnp.float32),
        pltpu.VMEM((3, c), jnp.int32),
        pltpu.VMEM_SHARED((n_nodes, o_dim), jnp.float32),
        pltpu.SemaphoreType.DMA((8,)),
        pltpu.SemaphoreType.DMA((4,)),
        pltpu.SemaphoreType.DMA((3,)),
    ]
    if with_cnt:
        out_type.append(jax.ShapeDtypeStruct((_NC, n_nodes, 16), jnp.float32))
        scratch += [
            pltpu.VMEM((c, 16), jnp.float32),
            pltpu.VMEM((zr, 16), jnp.float32),
            pltpu.VMEM_SHARED((n_nodes, 16), jnp.float32),
            pltpu.SemaphoreType.DMA((3,)),
        ]

    def body(xw, meta, *rest):
        if with_cnt:
            (agg_out, cnt_out, meta_m, rows_v, scv, dstc, agg_sh,
             sem_m, sem_g, sem_s,
             ones_v, z16_v, cnt_sh, sem_cs) = rest
        else:
            (agg_out, meta_m, rows_v, scv, dstc, agg_sh,
             sem_m, sem_g, sem_s) = rest
        cid = lax.axis_index("c")
        sid = lax.axis_index("s")
        wid = cid * _NT + sid
        row0 = sid * rpt

        @pl.loop(0, zr)
        def _fill_z(i):
            for j in range(o_dim // 16):
                scv[0, i, pl.ds(j * 16, 16)] = jnp.zeros((16,), jnp.float32)

        for r in range(rpt // zr):
            pltpu.sync_copy(scv.at[0], agg_sh.at[pl.ds(row0 + r * zr, zr)])

        if with_cnt:
            @pl.loop(0, zr)
            def _fill_z16(i):
                z16_v[i, :] = jnp.zeros((16,), jnp.float32)

            @pl.loop(0, c)
            def _fill_ones(i):
                ones_v[i, :] = jnp.ones((16,), jnp.float32)

            for r in range(rpt // zr):
                pltpu.sync_copy(z16_v, cnt_sh.at[pl.ds(row0 + r * zr, zr)])

        plsc.subcore_barrier()

        ebase = wid * ept

        def start_meta(g, b8):
            pltpu.async_copy(meta.at[:, pl.ds(ebase + g * c, c)],
                             meta_m.at[b8], sem_m.at[b8])

        def wait_meta(b8):
            pltpu.make_async_copy(meta.at[:, pl.ds(0, c)],
                                  meta_m.at[b8], sem_m.at[b8]).wait()

        def start_gather(b8, b3):
            pltpu.async_copy(xw.at[meta_m.at[b8, 0]], rows_v.at[b3],
                             sem_g.at[b3])

        def wait_gather(b3):
            pltpu.make_async_copy(xw.at[pl.ds(0, c)], rows_v.at[b3],
                                  sem_g.at[b3]).wait()

        def wait_scatter(b2):
            pltpu.make_async_copy(agg_out.at[0, pl.ds(0, c)], scv.at[b2],
                                  sem_s.at[b2]).wait()

        def wait_cnt_scatter(b2):
            pltpu.make_async_copy(cnt_out.at[0, pl.ds(0, c)], ones_v,
                                  sem_cs.at[b2]).wait()

        for k in range(8):
            start_meta(k, k)
        for k in range(3):
            wait_meta(k)
            start_gather(k, k)

        @pl.loop(0, nch)
        def _chunk(g):
            b2 = lax.rem(g, 3)
            b3 = lax.rem(g, 4)
            b8 = lax.rem(g, 8)
            wait_gather(b3)

            @pl.when(g >= 3)
            def _():
                wait_scatter(b2)
                if with_cnt:
                    wait_cnt_scatter(b2)

            @plsc.parallel_loop(0, c // 16)
            def _combine(grp):
                sl16 = pl.ds(grp * 16, 16)
                dstc[b2, sl16] = meta_m[b8, 5, sl16]
                s0 = plsc.bitcast(meta_m[b8, 1, sl16], jnp.float32)
                s1 = plsc.bitcast(meta_m[b8, 2, sl16], jnp.float32)
                s2 = plsc.bitcast(meta_m[b8, 3, sl16], jnp.float32)
                s3 = plsc.bitcast(meta_m[b8, 4, sl16], jnp.float32)
                vs = (s0, s1, s2, s3)
                for lane in range(16):
                    row = grp * 16 + lane
                    for m in range(o_dim // 32):
                        acc_a = None
                        acc_b = None
                        for s in range(4):
                            pk = rows_v[b3, row,
                                        pl.ds(s * o_dim + m * 32, 32)]
                            ua, ub = plsc.unpack(
                                pk, format=plsc.PackFormat.INTERLEAVED,
                                preferred_element_type=jnp.float32)
                            sval = vs[s][lane]
                            if acc_a is None:
                                acc_a = ua * sval
                                acc_b = ub * sval
                            else:
                                acc_a = acc_a + ua * sval
                                acc_b = acc_b + ub * sval
                        scv[b2, row, pl.ds(m * 32, 16)] = acc_a
                        scv[b2, row, pl.ds(m * 32 + 16, 16)] = acc_b

            pltpu.async_copy(scv.at[b2], agg_sh.at[dstc.at[b2]],
                             sem_s.at[b2], add=True)
            if with_cnt:
                pltpu.async_copy(ones_v, cnt_sh.at[dstc.at[b2]],
                                 sem_cs.at[b2], add=True)

            @pl.when(g + 8 < nch)
            def _():
                start_meta(g + 8, b8)

            @pl.when(g + 3 < nch)
            def _():
                b8n = lax.rem(g + 3, 8)
                b3n = lax.rem(g + 3, 4)
                wait_meta(b8n)
                start_gather(b8n, b3n)

        for k in range(3):
            wait_scatter(lax.rem(nch - 3 + k, 3))
            if with_cnt:
                wait_cnt_scatter(lax.rem(nch - 3 + k, 3))

        plsc.subcore_barrier()

        for r in range(rpt // zr):
            sl = pl.ds(row0 + r * zr, zr)
            pltpu.sync_copy(agg_sh.at[sl], scv.at[0])
            pltpu.sync_copy(scv.at[0], agg_out.at[cid, sl])
        if with_cnt:
            for r in range(rpt // zr):
                sl = pl.ds(row0 + r * zr, zr)
                pltpu.sync_copy(cnt_sh.at[sl], z16_v)
                pltpu.sync_copy(z16_v, cnt_out.at[cid, sl])

    if not with_cnt:
        out_type = out_type[0]
    return pl.kernel(
        body, out_type, mesh=mesh, scratch_types=scratch,
        compiler_params=pltpu.CompilerParams(use_tc_tiling_on_sc=False,
                                             needs_layout_passes=False))


def _post1_body(agg_ref, cnt_ref, xp_ref, root_ref, b_ref, w2_ref,
                h_ref, xw2_ref):
    a = agg_ref[0] + agg_ref[1]
    cnt = cnt_ref[0, :, 0:1] + cnt_ref[1, :, 0:1]
    t = (a / jnp.maximum(cnt, 1.0)
         + jax.lax.dot_general(xp_ref[...], root_ref[...],
                               (((1,), (0,)), ((), ())),
                               preferred_element_type=jnp.float32)
         + b_ref[0:1, :])
    h = jnp.where(t > 0, t, jnp.exp(t) - 1.0)
    h_ref[...] = h
    xw2_ref[...] = jax.lax.dot_general(
        h, w2_ref[...], (((1,), (0,)), ((), ())),
        preferred_element_type=jnp.float32).astype(jnp.bfloat16)


def _post1(agg1, cnt, xp, root1p, b1b, w2t, block_rows=1024):
    n = xp.shape[0]
    kd = w2t.shape[1]
    grid = n // block_rows
    return pl.pallas_call(
        _post1_body,
        grid=(grid,),
        in_specs=[
            pl.BlockSpec((2, block_rows, 32), lambda i: (0, i, 0)),
            pl.BlockSpec((2, block_rows, 16), lambda i: (0, i, 0)),
            pl.BlockSpec((block_rows, 8), lambda i: (i, 0)),
            pl.BlockSpec((8, 32), lambda i: (0, 0)),
            pl.BlockSpec((8, 32), lambda i: (0, 0)),
            pl.BlockSpec((32, kd), lambda i: (0, 0)),
        ],
        out_specs=[
            pl.BlockSpec((block_rows, 32), lambda i: (i, 0)),
            pl.BlockSpec((block_rows, kd), lambda i: (i, 0)),
        ],
        out_shape=[
            jax.ShapeDtypeStruct((n, 32), jnp.float32),
            jax.ShapeDtypeStruct((n, kd), jnp.bfloat16),
        ],
    )(agg1, cnt, xp, root1p, b1b, w2t)


def _post2_body(agg_ref, cnt_ref, h1_ref, root_ref, b_ref, o_ref, *,
                block_rows, n_real):
    a = agg_ref[0] + agg_ref[1]
    cnt = cnt_ref[0, :, 0:1] + cnt_ref[1, :, 0:1]
    t = (a / jnp.maximum(cnt, 1.0)
         + jax.lax.dot_general(h1_ref[...], root_ref[...],
                               (((1,), (0,)), ((), ())),
                               preferred_element_type=jnp.float32)
         + b_ref[0:1, :])
    h2 = jnp.where(t > 0, t, jnp.exp(t) - 1.0)
    row = (pl.program_id(0) * block_rows
           + jax.lax.broadcasted_iota(jnp.int32, (block_rows, 1), 0))
    h2 = jnp.where(row < n_real, h2, 0.0)

    @pl.when(pl.program_id(0) == 0)
    def _():
        o_ref[...] = jnp.zeros_like(o_ref)

    o_ref[0:1, :] += jnp.sum(h2, axis=0, keepdims=True)


def _post2(agg2, cnt, h1, root2, b2b, n_real, block_rows=1024):
    n = h1.shape[0]
    grid = n // block_rows
    return pl.pallas_call(
        functools.partial(_post2_body, block_rows=block_rows, n_real=n_real),
        grid=(grid,),
        in_specs=[
            pl.BlockSpec((2, block_rows, 64), lambda i: (0, i, 0)),
            pl.BlockSpec((2, block_rows, 16), lambda i: (0, i, 0)),
            pl.BlockSpec((block_rows, 32), lambda i: (i, 0)),
            pl.BlockSpec((32, 64), lambda i: (0, 0)),
            pl.BlockSpec((8, 64), lambda i: (0, 0)),
        ],
        out_specs=pl.BlockSpec((8, 64), lambda i: (0, 0)),
        out_shape=jax.ShapeDtypeStruct((8, 64), jnp.float32),
    )(agg2, cnt, h1, root2, b2b)


def _final_body(g_ref, lw1_ref, lb1_ref, lw2_ref, lb2_ref, o_ref, *, n):
    g = jnp.sum(g_ref[...], axis=0, keepdims=True) * (1.0 / n)
    g8 = jnp.broadcast_to(g, (8, 64))
    t = jax.lax.dot_general(g8, lw1_ref[...], (((1,), (0,)), ((), ())),
                            preferred_element_type=jnp.float32)
    t = t + lb1_ref[0:1, :]
    t = jnp.where(t > 0, t, jnp.exp(t) - 1.0)
    lg = jax.lax.dot_general(t, lw2_ref[...], (((1,), (0,)), ((), ())),
                             preferred_element_type=jnp.float32)
    lg = lg + lb2_ref[0:1, :]
    l0 = lg[0:1, 0:1]
    shifted = l0 - l0
    res = shifted - jnp.log(jnp.sum(jnp.exp(shifted)))
    o_ref[...] = jnp.broadcast_to(res, (8, 128))


def _final(gsum8, lw1, lb1b, lw2p, lb2b, n):
    return pl.pallas_call(
        functools.partial(_final_body, n=n),
        out_shape=jax.ShapeDtypeStruct((8, 128), jnp.float32),
    )(gsum8, lw1, lb1b, lw2p, lb2b)


def kernel(x, edge_index, edge_attr, batch, W1, root1, b1, W2, root2, b2,
           lw1, lb1, lw2, lb2):
    n = x.shape[0]
    e = edge_index.shape[1]
    cb = 128
    r = e // cb

    src2 = edge_index[0].reshape(r, cb)
    dst = edge_index[1]
    dst2 = dst.reshape(r, cb)
    eax2 = edge_attr[:, 0].reshape(r, cb)
    eay2 = edge_attr[:, 1].reshape(r, cb)

    meta = _edge_prep(src2, dst2, eax2, eay2).reshape(6, e)

    npad = 10240
    xp = jnp.pad(x, ((0, npad - n), (0, 5)))
    w1c = jnp.pad(_cell_weights(W1), ((0, 5), (0, 0)))
    xw1 = _matmul(xp, w1c, 1024).reshape(npad * 16, 128)

    agg1, cnt = _make_sc_pass(npad, 32, e, True)(xw1, meta)

    root1p = jnp.pad(root1, ((0, 5), (0, 0)))
    b1b = jnp.broadcast_to(b1.reshape(1, 32), (8, 32))
    w2c = _cell_weights(W2)
    h1, xw2 = _post1(agg1, cnt, xp, root1p, b1b, w2c)
    xw2 = xw2.reshape(npad * 16, 256)

    agg2 = _make_sc_pass(npad, 64, e, False)(xw2, meta)

    b2b = jnp.broadcast_to(b2.reshape(1, 64), (8, 64))
    gsum8 = _post2(agg2, cnt, h1, root2, b2b, n)

    lb1b = jnp.broadcast_to(lb1.reshape(1, 128), (8, 128))
    lw2p = jnp.pad(lw2, ((0, 0), (0, 7)))
    lb2b = jnp.broadcast_to(lb2.reshape(1, 1), (8, 8))
    out = _final(gsum8, lw1, lb1b, lw2p, lb2b, n)
    return out[:1, :1]
